# baseline (speedup 1.0000x reference)
import functools

import jax
import jax.numpy as jnp
from jax import lax
from jax.experimental import pallas as pl
from jax.experimental.pallas import tpu as pltpu
from jax.experimental.pallas import tpu_sc as plsc

NC = 2
NS = 16
NW = NC * NS
LANES = 16
CHUNK = 128


def _sc_aggregate(eall, ent_tab, rel_tab_hbm, S, B, H, EP, E):
  nchunk = EP // CHUNK
  nvec = H // LANES
  rel_rows = rel_tab_hbm.shape[0]
  inv_e = 1.0 / float(E)
  idxseg = nchunk * 3 * CHUNK
  mesh = plsc.VectorSubcoreMesh(core_axis_name="c", subcore_axis_name="s")

  @functools.partial(
      pl.kernel,
      out_type=jax.ShapeDtypeStruct((S * B * 3 * H,), jnp.float32),
      mesh=mesh,
      scratch_types=[
          pltpu.VMEM((idxseg,), jnp.int32),
          pltpu.VMEM((CHUNK, H), jnp.float32),
          pltpu.VMEM((CHUNK, H), jnp.float32),
          pltpu.VMEM((CHUNK, H), jnp.float32),
          pltpu.VMEM((CHUNK, H), jnp.float32),
          pltpu.VMEM((CHUNK, H), jnp.float32),
          pltpu.VMEM((CHUNK, H), jnp.float32),
          pltpu.VMEM((S * 3 * H,), jnp.float32),
          pltpu.SemaphoreType.DMA,
          pltpu.SemaphoreType.DMA,
      ],
  )
  def k(eall_hbm, ent_hbm, rel_hbm, out_hbm,
        idxb, sbufP, dbufP, rbufP, sbufQ, dbufQ, rbufQ, res, semP, semQ):
    wid = lax.axis_index("s") * NC + lax.axis_index("c")
    zvec = jnp.zeros((LANES,), jnp.float32)

    def gather(c, sb, db, rb, sem):
      coff = c * (3 * CHUNK)
      pltpu.async_copy(ent_hbm.at[idxb.at[pl.ds(coff, CHUNK)]], sb, sem)
      pltpu.async_copy(
          ent_hbm.at[idxb.at[pl.ds(coff + CHUNK, CHUNK)]], db, sem)
      pltpu.async_copy(
          rel_hbm.at[idxb.at[pl.ds(coff + 2 * CHUNK, CHUNK)]], rb, sem)

    def wait3(sb, db, rb, sem):
      pltpu.make_async_copy(ent_hbm.at[pl.ds(0, CHUNK)], sb, sem).wait()
      pltpu.make_async_copy(ent_hbm.at[pl.ds(0, CHUNK)], db, sem).wait()
      pltpu.make_async_copy(ent_hbm.at[pl.ds(0, CHUNK)], rb, sem).wait()

    def compute(sb, db, rb, acc):
      def edge_body(i, a):
        ea = list(a[0:nvec])
        ra = list(a[nvec:2 * nvec])
        wa = list(a[2 * nvec:3 * nvec])
        for v in range(nvec):
          sl = pl.ds(v * LANES, LANES)
          rv = rb[i, sl]
          sv = sb[i, sl]
          dv = db[i, sl]
          ea[v] = ea[v] + jnp.maximum(sv + rv, 0.0)
          wa[v] = wa[v] + jnp.maximum(dv + rv, 0.0)
          ra[v] = ra[v] + rv
        return tuple(ea) + tuple(ra) + tuple(wa)

      return lax.fori_loop(0, CHUNK, edge_body, acc, unroll=4)

    def seg_body(seg, carry):
      ibase = pl.multiple_of((wid * S + seg) * idxseg, CHUNK)
      pltpu.sync_copy(eall_hbm.at[pl.ds(ibase, idxseg)], idxb)
      gather(0, sbufP, dbufP, rbufP, semP)

      def pair_body(cp, acc):
        c0 = cp * 2
        c1 = c0 + 1
        gather(c1, sbufQ, dbufQ, rbufQ, semQ)
        wait3(sbufP, dbufP, rbufP, semP)
        acc = compute(sbufP, dbufP, rbufP, acc)

        @pl.when(c1 + 1 < nchunk)
        def _():
          gather(c1 + 1, sbufP, dbufP, rbufP, semP)

        wait3(sbufQ, dbufQ, rbufQ, semQ)
        return compute(sbufQ, dbufQ, rbufQ, acc)

      acc0 = (zvec,) * (3 * nvec)
      acc = lax.fori_loop(0, nchunk // 2, pair_body, acc0)
      for v in range(nvec):
        res[pl.ds(seg * 3 * H + v * LANES, LANES)] = acc[v] * inv_e
        res[pl.ds(seg * 3 * H + H + v * LANES, LANES)] = acc[nvec + v] * inv_e
        res[pl.ds(seg * 3 * H + 2 * H + v * LANES, LANES)] = (
            acc[2 * nvec + v] * inv_e)
      return carry

    lax.fori_loop(0, S, seg_body, 0)
    for s in range(S):
      dst_off = pl.multiple_of(s * (B * 3 * H) + wid * (3 * H), 3 * H)
      pltpu.sync_copy(res.at[pl.ds(s * 3 * H, 3 * H)],
                      out_hbm.at[pl.ds(dst_off, 3 * H)])

  return k(eall, ent_tab, rel_tab_hbm)


def _tc_head(embed, W_ih, W_hh, bih, bhh, wr, br, prob, tl, S, B, H):
  def body(embed_ref, wih_ref, whh_ref, bih_ref, bhh_ref, wr_ref, br_ref,
           prob_ref, tl_ref, out_ref):
    h = jnp.zeros((B, H), jnp.float32)
    wih = wih_ref[...]
    whh = whh_ref[...]
    bih_v = bih_ref[...]
    bhh_v = bhh_ref[...]
    for s in range(S):
      x = embed_ref[s]
      gi = jnp.dot(x, wih, preferred_element_type=jnp.float32) + bih_v
      gh = jnp.dot(h, whh, preferred_element_type=jnp.float32) + bhh_v
      r = jax.nn.sigmoid(gi[:, 0:H] + gh[:, 0:H])
      z = jax.nn.sigmoid(gi[:, H:2 * H] + gh[:, H:2 * H])
      n = jnp.tanh(gi[:, 2 * H:3 * H] + r * gh[:, 2 * H:3 * H])
      h = (1.0 - z) * n + z * h
    logit = jnp.sum(h * wr_ref[...], axis=1, keepdims=True) + br_ref[0, 0]
    pred = jax.nn.sigmoid(logit)
    ii = lax.broadcasted_iota(jnp.int32, (B, prob_ref.shape[1]), 1)
    tmat = jnp.where(ii == tl_ref[...], prob_ref[...], 0.0)
    target = jnp.sum(tmat, axis=1, keepdims=True)
    eps = 1e-7
    p = jnp.clip(pred, eps, 1.0 - eps)
    li = target * jnp.log(p) + (1.0 - target) * jnp.log(1.0 - p)
    out_ref[...] = jnp.reshape(-jnp.mean(li), (1, 1))

  return pl.pallas_call(
      body,
      out_shape=jax.ShapeDtypeStruct((1, 1), jnp.float32),
  )(embed, W_ih, W_hh, bih, bhh, wr, br, prob, tl)


def kernel(t_list, true_prob_r, edge_src, edge_dst, edge_rel,
           ent_embeds, rel_embeds, W_ih, W_hh, b_ih, b_hh, W_r, b_r):
  B, S, E = edge_src.shape
  H = ent_embeds.shape[1]
  num_ents = ent_embeds.shape[0]
  num_rels = rel_embeds.shape[0]
  EP = ((E + CHUNK - 1) // CHUNK) * CHUNK

  ent2 = jnp.concatenate(
      [ent_embeds, jnp.zeros((8, H), jnp.float32)], axis=0)
  rel2 = jnp.concatenate(
      [rel_embeds, jnp.zeros((8, H), jnp.float32)], axis=0)

  nchunk = EP // CHUNK

  def pad_edges(e, fill):
    e2 = e.reshape(B * S, E).astype(jnp.int32)
    pad = jnp.full((B * S, EP - E), fill, jnp.int32)
    return jnp.concatenate([e2, pad], axis=1).reshape(B * S, nchunk, CHUNK)

  esrc = pad_edges(edge_src, num_ents)
  edst = pad_edges(edge_dst, num_ents)
  erel = pad_edges(edge_rel, num_rels)
  eall = jnp.stack([esrc, edst, erel], axis=2).reshape(-1)

  embed_flat = _sc_aggregate(eall, ent2, rel2, S, B, H, EP, E)
  embed = embed_flat.reshape(S, B, 3 * H)

  T = true_prob_r.shape[0]
  TP = ((T + H - 1) // H) * H
  prob = jnp.concatenate(
      [true_prob_r, jnp.zeros((TP - T,), jnp.float32)]).reshape(1, TP)
  tl = t_list.astype(jnp.int32).reshape(B, 1)

  loss = _tc_head(embed, W_ih, W_hh,
                  b_ih.reshape(1, 3 * H), b_hh.reshape(1, 3 * H),
                  W_r.reshape(1, H), b_r.reshape(1, 1),
                  prob, tl, S, B, H)
  return loss[0, 0]


# speedup vs baseline: 1.8910x; 1.8910x over previous
import functools

import jax
import jax.numpy as jnp
from jax import lax
from jax.experimental import pallas as pl
from jax.experimental.pallas import tpu as pltpu
from jax.experimental.pallas import tpu_sc as plsc

NC = 2
NS = 16
NW = NC * NS
LANES = 16
CHUNK = 128


def _sc_aggregate(eall, ent_tab, rel_tab_hbm, S, B, H, EP, E):
  nchunk = EP // CHUNK
  nvec = H // LANES
  rel_rows = rel_tab_hbm.shape[0]
  inv_e = 1.0 / float(E)
  idxseg = nchunk * 3 * CHUNK
  mesh = plsc.VectorSubcoreMesh(core_axis_name="c", subcore_axis_name="s")

  @functools.partial(
      pl.kernel,
      out_type=jax.ShapeDtypeStruct((S * B * 3 * H,), jnp.float32),
      mesh=mesh,
      compiler_params=pltpu.CompilerParams(
          needs_layout_passes=False, use_tc_tiling_on_sc=False),
      scratch_types=[
          pltpu.VMEM((idxseg,), jnp.int32),
          pltpu.VMEM((CHUNK, H // 2), jnp.int32),
          pltpu.VMEM((CHUNK, H // 2), jnp.int32),
          pltpu.VMEM((CHUNK, H // 2), jnp.int32),
          pltpu.VMEM((CHUNK, H // 2), jnp.int32),
          pltpu.VMEM((CHUNK, H // 2), jnp.int32),
          pltpu.VMEM((CHUNK, H // 2), jnp.int32),
          pltpu.VMEM((S * 3 * H,), jnp.float32),
          pltpu.SemaphoreType.DMA,
          pltpu.SemaphoreType.DMA,
      ],
  )
  def k(eall_hbm, ent_hbm, rel_hbm, out_hbm,
        idxb, sbufP, dbufP, rbufP, sbufQ, dbufQ, rbufQ, res, semP, semQ):
    wid = lax.axis_index("s") * NC + lax.axis_index("c")
    zvec = jnp.zeros((LANES,), jnp.float32)
    nv2 = H // (2 * LANES)

    def gather(c, sb, db, rb, sem):
      coff = c * (3 * CHUNK)
      pltpu.async_copy(ent_hbm.at[idxb.at[pl.ds(coff, CHUNK)]], sb, sem)
      pltpu.async_copy(
          ent_hbm.at[idxb.at[pl.ds(coff + CHUNK, CHUNK)]], db, sem)
      pltpu.async_copy(
          rel_hbm.at[idxb.at[pl.ds(coff + 2 * CHUNK, CHUNK)]], rb, sem)

    def wait3(sb, db, rb, sem):
      pltpu.make_async_copy(ent_hbm.at[pl.ds(0, CHUNK)], sb, sem).wait()
      pltpu.make_async_copy(ent_hbm.at[pl.ds(0, CHUNK)], db, sem).wait()
      pltpu.make_async_copy(ent_hbm.at[pl.ds(0, CHUNK)], rb, sem).wait()

    def compute(sb, db, rb, acc):
      zb = jnp.zeros((2 * LANES,), jnp.bfloat16)

      def edge_body(i, a):
        a = list(a)
        for v in range(nv2):
          sl = pl.ds(v * LANES, LANES)
          rv = plsc.bitcast(rb[i, sl], jnp.bfloat16)
          sv = plsc.bitcast(sb[i, sl], jnp.bfloat16)
          dv = plsc.bitcast(db[i, sl], jnp.bfloat16)
          m = jnp.maximum(sv + rv, zb)
          w = jnp.maximum(dv + rv, zb)
          ma, mb = plsc.unpack(m, format=plsc.PackFormat.INTERLEAVED)
          wa, wb = plsc.unpack(w, format=plsc.PackFormat.INTERLEAVED)
          ca, cb = plsc.unpack(rv, format=plsc.PackFormat.INTERLEAVED)
          a[2 * v] += ma
          a[2 * v + 1] += mb
          a[nvec + 2 * v] += ca
          a[nvec + 2 * v + 1] += cb
          a[2 * nvec + 2 * v] += wa
          a[2 * nvec + 2 * v + 1] += wb
        return tuple(a)

      return lax.fori_loop(0, CHUNK, edge_body, acc, unroll=2)

    def seg_body(seg, carry):
      ibase = pl.multiple_of((wid * S + seg) * idxseg, CHUNK)
      pltpu.sync_copy(eall_hbm.at[pl.ds(ibase, idxseg)], idxb)
      gather(0, sbufP, dbufP, rbufP, semP)

      def pair_body(cp, acc):
        c0 = cp * 2
        c1 = c0 + 1
        gather(c1, sbufQ, dbufQ, rbufQ, semQ)
        wait3(sbufP, dbufP, rbufP, semP)
        acc = compute(sbufP, dbufP, rbufP, acc)

        @pl.when(c1 + 1 < nchunk)
        def _():
          gather(c1 + 1, sbufP, dbufP, rbufP, semP)

        wait3(sbufQ, dbufQ, rbufQ, semQ)
        return compute(sbufQ, dbufQ, rbufQ, acc)

      acc0 = (zvec,) * (3 * nvec)
      acc = lax.fori_loop(0, nchunk // 2, pair_body, acc0)
      ii2 = 2 * lax.iota(jnp.int32, LANES)
      for pool in range(3):
        for v in range(nv2):
          base = seg * 3 * H + pool * H + v * 2 * LANES
          plsc.store_scatter(res, [base + ii2],
                             acc[pool * nvec + 2 * v] * inv_e)
          plsc.store_scatter(res, [base + ii2 + 1],
                             acc[pool * nvec + 2 * v + 1] * inv_e)
      return carry

    lax.fori_loop(0, S, seg_body, 0)
    for s in range(S):
      dst_off = pl.multiple_of(s * (B * 3 * H) + wid * (3 * H), 3 * H)
      pltpu.sync_copy(res.at[pl.ds(s * 3 * H, 3 * H)],
                      out_hbm.at[pl.ds(dst_off, 3 * H)])

  return k(eall, ent_tab, rel_tab_hbm)


def _tc_head(embed, W_ih, W_hh, bih, bhh, wr, br, prob, tl, S, B, H):
  def body(embed_ref, wih_ref, whh_ref, bih_ref, bhh_ref, wr_ref, br_ref,
           prob_ref, tl_ref, out_ref):
    h = jnp.zeros((B, H), jnp.float32)
    wih = wih_ref[...]
    whh = whh_ref[...]
    bih_v = bih_ref[...]
    bhh_v = bhh_ref[...]
    for s in range(S):
      x = embed_ref[s]
      gi = jnp.dot(x, wih, preferred_element_type=jnp.float32) + bih_v
      gh = jnp.dot(h, whh, preferred_element_type=jnp.float32) + bhh_v
      r = jax.nn.sigmoid(gi[:, 0:H] + gh[:, 0:H])
      z = jax.nn.sigmoid(gi[:, H:2 * H] + gh[:, H:2 * H])
      n = jnp.tanh(gi[:, 2 * H:3 * H] + r * gh[:, 2 * H:3 * H])
      h = (1.0 - z) * n + z * h
    logit = jnp.sum(h * wr_ref[...], axis=1, keepdims=True) + br_ref[0, 0]
    pred = jax.nn.sigmoid(logit)
    ii = lax.broadcasted_iota(jnp.int32, (B, prob_ref.shape[1]), 1)
    tmat = jnp.where(ii == tl_ref[...], prob_ref[...], 0.0)
    target = jnp.sum(tmat, axis=1, keepdims=True)
    eps = 1e-7
    p = jnp.clip(pred, eps, 1.0 - eps)
    li = target * jnp.log(p) + (1.0 - target) * jnp.log(1.0 - p)
    out_ref[...] = jnp.reshape(-jnp.mean(li), (1, 1))

  return pl.pallas_call(
      body,
      out_shape=jax.ShapeDtypeStruct((1, 1), jnp.float32),
  )(embed, W_ih, W_hh, bih, bhh, wr, br, prob, tl)


def kernel(t_list, true_prob_r, edge_src, edge_dst, edge_rel,
           ent_embeds, rel_embeds, W_ih, W_hh, b_ih, b_hh, W_r, b_r):
  B, S, E = edge_src.shape
  H = ent_embeds.shape[1]
  num_ents = ent_embeds.shape[0]
  num_rels = rel_embeds.shape[0]
  EP = ((E + CHUNK - 1) // CHUNK) * CHUNK

  def to_words(tab):
    b = jnp.concatenate(
        [tab, jnp.zeros((8, H), jnp.float32)], axis=0).astype(jnp.bfloat16)
    return lax.bitcast_convert_type(
        b.reshape(b.shape[0], H // 2, 2), jnp.int32)

  ent2 = to_words(ent_embeds)
  rel2 = to_words(rel_embeds)

  nchunk = EP // CHUNK

  def pad_edges(e, fill):
    e2 = e.reshape(B * S, E).astype(jnp.int32)
    pad = jnp.full((B * S, EP - E), fill, jnp.int32)
    return jnp.concatenate([e2, pad], axis=1).reshape(B * S, nchunk, CHUNK)

  esrc = pad_edges(edge_src, num_ents)
  edst = pad_edges(edge_dst, num_ents)
  erel = pad_edges(edge_rel, num_rels)
  eall = jnp.stack([esrc, edst, erel], axis=2).reshape(-1)

  embed_flat = _sc_aggregate(eall, ent2, rel2, S, B, H, EP, E)
  embed = embed_flat.reshape(S, B, 3 * H)

  T = true_prob_r.shape[0]
  TP = ((T + H - 1) // H) * H
  prob = jnp.concatenate(
      [true_prob_r, jnp.zeros((TP - T,), jnp.float32)]).reshape(1, TP)
  tl = t_list.astype(jnp.int32).reshape(B, 1)

  loss = _tc_head(embed, W_ih, W_hh,
                  b_ih.reshape(1, 3 * H), b_hh.reshape(1, 3 * H),
                  W_r.reshape(1, H), b_r.reshape(1, 1),
                  prob, tl, S, B, H)
  return loss[0, 0]


# speedup vs baseline: 2.0136x; 1.0649x over previous
import functools

import jax
import jax.numpy as jnp
from jax import lax
from jax.experimental import pallas as pl
from jax.experimental.pallas import tpu as pltpu
from jax.experimental.pallas import tpu_sc as plsc

NC = 2
NS = 16
NW = NC * NS
LANES = 16
CHUNK = 128


def _sc_aggregate(eall, ent_tab, rel_tab_hbm, S, B, H, EP, E):
  nchunk = EP // CHUNK
  nvec = H // LANES
  rel_rows = rel_tab_hbm.shape[0]
  inv_e = 1.0 / float(E)
  idxseg = nchunk * 3 * CHUNK
  mesh = plsc.VectorSubcoreMesh(core_axis_name="c", subcore_axis_name="s")

  @functools.partial(
      pl.kernel,
      out_type=jax.ShapeDtypeStruct((S * B * 3 * H,), jnp.float32),
      mesh=mesh,
      compiler_params=pltpu.CompilerParams(
          needs_layout_passes=False, use_tc_tiling_on_sc=False),
      scratch_types=[
          pltpu.VMEM((idxseg,), jnp.int32),
          pltpu.VMEM((rel_rows, H // 2), jnp.int32),
          pltpu.VMEM((CHUNK, H // 2), jnp.int32),
          pltpu.VMEM((CHUNK, H // 2), jnp.int32),
          pltpu.VMEM((CHUNK, H // 2), jnp.int32),
          pltpu.VMEM((CHUNK, H // 2), jnp.int32),
          pltpu.VMEM((S * 3 * H,), jnp.float32),
          pltpu.SemaphoreType.DMA,
          pltpu.SemaphoreType.DMA,
      ],
  )
  def k(eall_hbm, ent_hbm, rel_hbm, out_hbm,
        idxb, rtab, sbufP, dbufP, sbufQ, dbufQ, res, semP, semQ):
    wid = lax.axis_index("s") * NC + lax.axis_index("c")
    zvec = jnp.zeros((LANES,), jnp.float32)
    nv2 = H // (2 * LANES)
    pltpu.sync_copy(rel_hbm, rtab)

    def gather(c, sb, db, sem):
      coff = c * (3 * CHUNK)
      pltpu.async_copy(ent_hbm.at[idxb.at[pl.ds(coff, CHUNK)]], sb, sem)
      pltpu.async_copy(
          ent_hbm.at[idxb.at[pl.ds(coff + CHUNK, CHUNK)]], db, sem)

    def wait2(sb, db, sem):
      pltpu.make_async_copy(ent_hbm.at[pl.ds(0, CHUNK)], sb, sem).wait()
      pltpu.make_async_copy(ent_hbm.at[pl.ds(0, CHUNK)], db, sem).wait()

    def compute(c, sb, db, acc):
      zb = jnp.zeros((2 * LANES,), jnp.bfloat16)

      def group_body(j, a):
        a = list(a)
        rvec = idxb[pl.ds(c * (3 * CHUNK) + 2 * CHUNK + j * LANES, LANES)]
        for e in range(LANES):
          r = rvec[e]
          i = j * LANES + e
          for v in range(nv2):
            sl = pl.ds(v * LANES, LANES)
            rv = plsc.bitcast(rtab[r, sl], jnp.bfloat16)
            sv = plsc.bitcast(sb[i, sl], jnp.bfloat16)
            dv = plsc.bitcast(db[i, sl], jnp.bfloat16)
            m = jnp.maximum(sv + rv, zb)
            w = jnp.maximum(dv + rv, zb)
            ma, mb = plsc.unpack(m, format=plsc.PackFormat.INTERLEAVED)
            wa, wb = plsc.unpack(w, format=plsc.PackFormat.INTERLEAVED)
            ca, cb = plsc.unpack(rv, format=plsc.PackFormat.INTERLEAVED)
            a[2 * v] += ma
            a[2 * v + 1] += mb
            a[nvec + 2 * v] += ca
            a[nvec + 2 * v + 1] += cb
            a[2 * nvec + 2 * v] += wa
            a[2 * nvec + 2 * v + 1] += wb
        return tuple(a)

      return lax.fori_loop(0, CHUNK // LANES, group_body, acc)

    def seg_body(seg, carry):
      ibase = pl.multiple_of((wid * S + seg) * idxseg, CHUNK)
      pltpu.sync_copy(eall_hbm.at[pl.ds(ibase, idxseg)], idxb)
      gather(0, sbufP, dbufP, semP)

      def pair_body(cp, acc):
        c0 = cp * 2
        c1 = c0 + 1
        gather(c1, sbufQ, dbufQ, semQ)
        wait2(sbufP, dbufP, semP)
        acc = compute(c0, sbufP, dbufP, acc)

        @pl.when(c1 + 1 < nchunk)
        def _():
          gather(c1 + 1, sbufP, dbufP, semP)

        wait2(sbufQ, dbufQ, semQ)
        return compute(c1, sbufQ, dbufQ, acc)

      acc0 = (zvec,) * (3 * nvec)
      acc = lax.fori_loop(0, nchunk // 2, pair_body, acc0)
      ii2 = 2 * lax.iota(jnp.int32, LANES)
      for pool in range(3):
        for v in range(nv2):
          base = seg * 3 * H + pool * H + v * 2 * LANES
          plsc.store_scatter(res, [base + ii2],
                             acc[pool * nvec + 2 * v] * inv_e)
          plsc.store_scatter(res, [base + ii2 + 1],
                             acc[pool * nvec + 2 * v + 1] * inv_e)
      return carry

    lax.fori_loop(0, S, seg_body, 0)
    for s in range(S):
      dst_off = pl.multiple_of(s * (B * 3 * H) + wid * (3 * H), 3 * H)
      pltpu.sync_copy(res.at[pl.ds(s * 3 * H, 3 * H)],
                      out_hbm.at[pl.ds(dst_off, 3 * H)])

  return k(eall, ent_tab, rel_tab_hbm)


def _tc_head(embed, W_ih, W_hh, bih, bhh, wr, br, prob, tl, S, B, H):
  def body(embed_ref, wih_ref, whh_ref, bih_ref, bhh_ref, wr_ref, br_ref,
           prob_ref, tl_ref, out_ref):
    h = jnp.zeros((B, H), jnp.float32)
    wih = wih_ref[...]
    whh = whh_ref[...]
    bih_v = bih_ref[...]
    bhh_v = bhh_ref[...]
    for s in range(S):
      x = embed_ref[s]
      gi = jnp.dot(x, wih, preferred_element_type=jnp.float32) + bih_v
      gh = jnp.dot(h, whh, preferred_element_type=jnp.float32) + bhh_v
      r = jax.nn.sigmoid(gi[:, 0:H] + gh[:, 0:H])
      z = jax.nn.sigmoid(gi[:, H:2 * H] + gh[:, H:2 * H])
      n = jnp.tanh(gi[:, 2 * H:3 * H] + r * gh[:, 2 * H:3 * H])
      h = (1.0 - z) * n + z * h
    logit = jnp.sum(h * wr_ref[...], axis=1, keepdims=True) + br_ref[0, 0]
    pred = jax.nn.sigmoid(logit)
    ii = lax.broadcasted_iota(jnp.int32, (B, prob_ref.shape[1]), 1)
    tmat = jnp.where(ii == tl_ref[...], prob_ref[...], 0.0)
    target = jnp.sum(tmat, axis=1, keepdims=True)
    eps = 1e-7
    p = jnp.clip(pred, eps, 1.0 - eps)
    li = target * jnp.log(p) + (1.0 - target) * jnp.log(1.0 - p)
    out_ref[...] = jnp.reshape(-jnp.mean(li), (1, 1))

  return pl.pallas_call(
      body,
      out_shape=jax.ShapeDtypeStruct((1, 1), jnp.float32),
  )(embed, W_ih, W_hh, bih, bhh, wr, br, prob, tl)


def kernel(t_list, true_prob_r, edge_src, edge_dst, edge_rel,
           ent_embeds, rel_embeds, W_ih, W_hh, b_ih, b_hh, W_r, b_r):
  B, S, E = edge_src.shape
  H = ent_embeds.shape[1]
  num_ents = ent_embeds.shape[0]
  num_rels = rel_embeds.shape[0]
  EP = ((E + CHUNK - 1) // CHUNK) * CHUNK

  def to_words(tab):
    b = jnp.concatenate(
        [tab, jnp.zeros((8, H), jnp.float32)], axis=0).astype(jnp.bfloat16)
    return lax.bitcast_convert_type(
        b.reshape(b.shape[0], H // 2, 2), jnp.int32)

  ent2 = to_words(ent_embeds)
  rel2 = to_words(rel_embeds)

  nchunk = EP // CHUNK

  def pad_edges(e, fill):
    e2 = e.reshape(B * S, E).astype(jnp.int32)
    pad = jnp.full((B * S, EP - E), fill, jnp.int32)
    return jnp.concatenate([e2, pad], axis=1).reshape(B * S, nchunk, CHUNK)

  esrc = pad_edges(edge_src, num_ents)
  edst = pad_edges(edge_dst, num_ents)
  erel = pad_edges(edge_rel, num_rels)
  eall = jnp.stack([esrc, edst, erel], axis=2).reshape(-1)

  embed_flat = _sc_aggregate(eall, ent2, rel2, S, B, H, EP, E)
  embed = embed_flat.reshape(S, B, 3 * H)

  T = true_prob_r.shape[0]
  TP = ((T + H - 1) // H) * H
  prob = jnp.concatenate(
      [true_prob_r, jnp.zeros((TP - T,), jnp.float32)]).reshape(1, TP)
  tl = t_list.astype(jnp.int32).reshape(B, 1)

  loss = _tc_head(embed, W_ih, W_hh,
                  b_ih.reshape(1, 3 * H), b_hh.reshape(1, 3 * H),
                  W_r.reshape(1, H), b_r.reshape(1, 1),
                  prob, tl, S, B, H)
  return loss[0, 0]


# speedup vs baseline: 2.0432x; 1.0147x over previous
import functools

import jax
import jax.numpy as jnp
from jax import lax
from jax.experimental import pallas as pl
from jax.experimental.pallas import tpu as pltpu
from jax.experimental.pallas import tpu_sc as plsc

NC = 2
NS = 16
NW = NC * NS
LANES = 16
CHUNK = 128


def _sc_aggregate(eall, ent_tab, rel_tab_hbm, S, B, H, EP, E):
  nchunk = EP // CHUNK
  nvec = H // LANES
  rel_rows = rel_tab_hbm.shape[0]
  inv_e = 1.0 / float(E)
  idxseg = nchunk * 3 * CHUNK
  mesh = plsc.VectorSubcoreMesh(core_axis_name="c", subcore_axis_name="s")

  RR = 272

  @functools.partial(
      pl.kernel,
      out_type=(jax.ShapeDtypeStruct((S * B * 3 * H,), jnp.float32),
                jax.ShapeDtypeStruct((S * B * RR,), jnp.float32)),
      mesh=mesh,
      compiler_params=pltpu.CompilerParams(
          needs_layout_passes=False, use_tc_tiling_on_sc=False),
      scratch_types=[
          pltpu.VMEM((idxseg,), jnp.int32),
          pltpu.VMEM((rel_rows, H // 2), jnp.int32),
          pltpu.VMEM((272,), jnp.float32),
          pltpu.VMEM((CHUNK, H // 2), jnp.int32),
          pltpu.VMEM((CHUNK, H // 2), jnp.int32),
          pltpu.VMEM((CHUNK, H // 2), jnp.int32),
          pltpu.VMEM((CHUNK, H // 2), jnp.int32),
          pltpu.VMEM((S * 3 * H,), jnp.float32),
          pltpu.SemaphoreType.DMA,
          pltpu.SemaphoreType.DMA,
      ],
  )
  def k(eall_hbm, ent_hbm, rel_hbm, out_hbm, cnt_hbm,
        idxb, rtab, cnt, sbufP, dbufP, sbufQ, dbufQ, res, semP, semQ):
    wid = lax.axis_index("s") * NC + lax.axis_index("c")
    zvec = jnp.zeros((LANES,), jnp.float32)
    nv2 = H // (2 * LANES)
    ones = jnp.ones((LANES,), jnp.float32)
    pltpu.sync_copy(rel_hbm, rtab)

    def gather(c, sb, db, sem):
      coff = c * (3 * CHUNK)
      pltpu.async_copy(ent_hbm.at[idxb.at[pl.ds(coff, CHUNK)]], sb, sem)
      pltpu.async_copy(
          ent_hbm.at[idxb.at[pl.ds(coff + CHUNK, CHUNK)]], db, sem)

    def wait2(sb, db, sem):
      pltpu.make_async_copy(ent_hbm.at[pl.ds(0, CHUNK)], sb, sem).wait()
      pltpu.make_async_copy(ent_hbm.at[pl.ds(0, CHUNK)], db, sem).wait()

    def compute(c, sb, db, acc):
      zb = jnp.zeros((2 * LANES,), jnp.bfloat16)

      def group_body(j, a):
        a = list(a)
        rvec = idxb[pl.ds(c * (3 * CHUNK) + 2 * CHUNK + j * LANES, LANES)]
        plsc.addupdate_scatter(cnt, [rvec], ones)
        for e in range(LANES):
          r = rvec[e]
          i = j * LANES + e
          for v in range(nv2):
            sl = pl.ds(v * LANES, LANES)
            rv = plsc.bitcast(rtab[r, sl], jnp.bfloat16)
            sv = plsc.bitcast(sb[i, sl], jnp.bfloat16)
            dv = plsc.bitcast(db[i, sl], jnp.bfloat16)
            m = jnp.maximum(sv + rv, zb)
            w = jnp.maximum(dv + rv, zb)
            ma, mb = plsc.unpack(m, format=plsc.PackFormat.INTERLEAVED)
            wa, wb = plsc.unpack(w, format=plsc.PackFormat.INTERLEAVED)
            a[2 * v] += ma
            a[2 * v + 1] += mb
            a[nvec + 2 * v] += wa
            a[nvec + 2 * v + 1] += wb
        return tuple(a)

      return lax.fori_loop(0, CHUNK // LANES, group_body, acc)

    def seg_body(seg, carry):
      ibase = pl.multiple_of((wid * S + seg) * idxseg, CHUNK)
      pltpu.sync_copy(eall_hbm.at[pl.ds(ibase, idxseg)], idxb)
      gather(0, sbufP, dbufP, semP)

      def pair_body(cp, acc):
        c0 = cp * 2
        c1 = c0 + 1
        gather(c1, sbufQ, dbufQ, semQ)
        wait2(sbufP, dbufP, semP)
        acc = compute(c0, sbufP, dbufP, acc)

        @pl.when(c1 + 1 < nchunk)
        def _():
          gather(c1 + 1, sbufP, dbufP, semP)

        wait2(sbufQ, dbufQ, semQ)
        return compute(c1, sbufQ, dbufQ, acc)

      for t in range(RR // LANES):
        cnt[pl.ds(t * LANES, LANES)] = zvec
      acc0 = (zvec,) * (2 * nvec)
      acc = lax.fori_loop(0, nchunk // 2, pair_body, acc0)
      ii2 = 2 * lax.iota(jnp.int32, LANES)
      for pool in range(2):
        for v in range(nv2):
          base = seg * 3 * H + 2 * pool * H + v * 2 * LANES
          plsc.store_scatter(res, [base + ii2],
                             acc[pool * nvec + 2 * v] * inv_e)
          plsc.store_scatter(res, [base + ii2 + 1],
                             acc[pool * nvec + 2 * v + 1] * inv_e)
      for v in range(nvec):
        res[pl.ds(seg * 3 * H + H + v * LANES, LANES)] = zvec
      cbase = pl.multiple_of((seg * B + wid) * RR, LANES)
      pltpu.sync_copy(cnt, cnt_hbm.at[pl.ds(cbase, RR)])
      return carry

    lax.fori_loop(0, S, seg_body, 0)
    for s in range(S):
      dst_off = pl.multiple_of(s * (B * 3 * H) + wid * (3 * H), 3 * H)
      pltpu.sync_copy(res.at[pl.ds(s * 3 * H, 3 * H)],
                      out_hbm.at[pl.ds(dst_off, 3 * H)])

  return k(eall, ent_tab, rel_tab_hbm)


def _tc_head(embed, cnt3, rtabf, W_ih, W_hh, bih, bhh, wr, br, prob, tl,
             S, B, H, E):
  inv_e = 1.0 / float(E)

  def body(embed_ref, cnt_ref, rtab_ref, wih_ref, whh_ref, bih_ref, bhh_ref,
           wr_ref, br_ref, prob_ref, tl_ref, out_ref):
    h = jnp.zeros((B, H), jnp.float32)
    wih = wih_ref[...]
    whh = whh_ref[...]
    bih_v = bih_ref[...]
    bhh_v = bhh_ref[...]
    rtw = jnp.dot(rtab_ref[...], wih[H:2 * H, :],
                  preferred_element_type=jnp.float32) * inv_e
    for s in range(S):
      x = embed_ref[s]
      gi = (jnp.dot(x, wih, preferred_element_type=jnp.float32)
            + jnp.dot(cnt_ref[s], rtw, preferred_element_type=jnp.float32)
            + bih_v)
      gh = jnp.dot(h, whh, preferred_element_type=jnp.float32) + bhh_v
      r = jax.nn.sigmoid(gi[:, 0:H] + gh[:, 0:H])
      z = jax.nn.sigmoid(gi[:, H:2 * H] + gh[:, H:2 * H])
      n = jnp.tanh(gi[:, 2 * H:3 * H] + r * gh[:, 2 * H:3 * H])
      h = (1.0 - z) * n + z * h
    logit = jnp.sum(h * wr_ref[...], axis=1, keepdims=True) + br_ref[0, 0]
    pred = jax.nn.sigmoid(logit)
    ii = lax.broadcasted_iota(jnp.int32, (B, prob_ref.shape[1]), 1)
    tmat = jnp.where(ii == tl_ref[...], prob_ref[...], 0.0)
    target = jnp.sum(tmat, axis=1, keepdims=True)
    eps = 1e-7
    p = jnp.clip(pred, eps, 1.0 - eps)
    li = target * jnp.log(p) + (1.0 - target) * jnp.log(1.0 - p)
    out_ref[...] = jnp.reshape(-jnp.mean(li), (1, 1))

  return pl.pallas_call(
      body,
      out_shape=jax.ShapeDtypeStruct((1, 1), jnp.float32),
  )(embed, cnt3, rtabf, W_ih, W_hh, bih, bhh, wr, br, prob, tl)


def kernel(t_list, true_prob_r, edge_src, edge_dst, edge_rel,
           ent_embeds, rel_embeds, W_ih, W_hh, b_ih, b_hh, W_r, b_r):
  B, S, E = edge_src.shape
  H = ent_embeds.shape[1]
  num_ents = ent_embeds.shape[0]
  num_rels = rel_embeds.shape[0]
  EP = ((E + CHUNK - 1) // CHUNK) * CHUNK

  def to_words(tab):
    b = jnp.concatenate(
        [tab, jnp.zeros((8, H), jnp.float32)], axis=0).astype(jnp.bfloat16)
    return lax.bitcast_convert_type(
        b.reshape(b.shape[0], H // 2, 2), jnp.int32)

  ent2 = to_words(ent_embeds)
  rel2 = to_words(rel_embeds)

  nchunk = EP // CHUNK

  def pad_edges(e, fill):
    e2 = e.reshape(B * S, E).astype(jnp.int32)
    pad = jnp.full((B * S, EP - E), fill, jnp.int32)
    return jnp.concatenate([e2, pad], axis=1).reshape(B * S, nchunk, CHUNK)

  esrc = pad_edges(edge_src, num_ents)
  edst = pad_edges(edge_dst, num_ents)
  erel = pad_edges(edge_rel, num_rels)
  eall = jnp.stack([esrc, edst, erel], axis=2).reshape(-1)

  embed_flat, cnt_flat = _sc_aggregate(eall, ent2, rel2, S, B, H, EP, E)
  embed = embed_flat.reshape(S, B, 3 * H)
  RR = 272
  cnt3 = cnt_flat.reshape(S, B, RR)
  rtabf = jnp.concatenate(
      [rel_embeds, jnp.zeros((RR - num_rels, H), jnp.float32)], axis=0)

  T = true_prob_r.shape[0]
  TP = ((T + H - 1) // H) * H
  prob = jnp.concatenate(
      [true_prob_r, jnp.zeros((TP - T,), jnp.float32)]).reshape(1, TP)
  tl = t_list.astype(jnp.int32).reshape(B, 1)

  loss = _tc_head(embed, cnt3, rtabf, W_ih, W_hh,
                  b_ih.reshape(1, 3 * H), b_hh.reshape(1, 3 * H),
                  W_r.reshape(1, H), b_r.reshape(1, 1),
                  prob, tl, S, B, H, E)
  return loss[0, 0]


# speedup vs baseline: 2.0504x; 1.0035x over previous
import functools

import jax
import jax.numpy as jnp
from jax import lax
from jax.experimental import pallas as pl
from jax.experimental.pallas import tpu as pltpu
from jax.experimental.pallas import tpu_sc as plsc

NC = 2
NS = 16
NW = NC * NS
LANES = 16
CHUNK = 128


def _sc_aggregate(eall, ent_tab, rel_tab_hbm, S, B, H, EP, E):
  nchunk = EP // CHUNK
  nvec = H // LANES
  rel_rows = rel_tab_hbm.shape[0]
  inv_e = 1.0 / float(E)
  idxseg = nchunk * 3 * CHUNK
  mesh = plsc.VectorSubcoreMesh(core_axis_name="c", subcore_axis_name="s")

  RR = 272

  @functools.partial(
      pl.kernel,
      out_type=(jax.ShapeDtypeStruct((S * B * 3 * H,), jnp.float32),
                jax.ShapeDtypeStruct((S * B * RR,), jnp.float32)),
      mesh=mesh,
      compiler_params=pltpu.CompilerParams(
          needs_layout_passes=False, use_tc_tiling_on_sc=False),
      scratch_types=[
          pltpu.VMEM((S * nchunk * 3 * CHUNK,), jnp.int32),
          pltpu.VMEM((rel_rows, H // 2), jnp.int32),
          pltpu.VMEM((272,), jnp.float32),
          pltpu.VMEM((2 * CHUNK, H // 2), jnp.int32),
          pltpu.VMEM((2 * CHUNK, H // 2), jnp.int32),
          pltpu.VMEM((S * 3 * H,), jnp.float32),
          pltpu.SemaphoreType.DMA,
          pltpu.SemaphoreType.DMA,
      ],
  )
  def k(eall_hbm, ent_hbm, rel_hbm, out_hbm, cnt_hbm,
        idxb, rtab, cnt, bufP, bufQ, res, semP, semQ):
    wid = lax.axis_index("s") * NC + lax.axis_index("c")
    zvec = jnp.zeros((LANES,), jnp.float32)
    nv2 = H // (2 * LANES)
    ones = jnp.ones((LANES,), jnp.float32)
    pltpu.sync_copy(rel_hbm, rtab)
    ibase = pl.multiple_of(wid * S * idxseg, CHUNK)
    pltpu.sync_copy(eall_hbm.at[pl.ds(ibase, S * idxseg)], idxb)

    def gather(seg, c, buf, sem):
      coff = seg * idxseg + c * (3 * CHUNK)
      pltpu.async_copy(ent_hbm.at[idxb.at[pl.ds(coff, CHUNK)]],
                       buf.at[pl.ds(0, CHUNK)], sem)
      pltpu.async_copy(ent_hbm.at[idxb.at[pl.ds(coff + CHUNK, CHUNK)]],
                       buf.at[pl.ds(CHUNK, CHUNK)], sem)

    def wait1(buf, sem):
      pltpu.make_async_copy(ent_hbm.at[pl.ds(0, 2 * CHUNK)], buf, sem).wait()

    def compute(seg, c, buf, acc):
      zb = jnp.zeros((2 * LANES,), jnp.bfloat16)

      def group_body(j, a):
        a = list(a)
        rvec = idxb[pl.ds(seg * idxseg + c * (3 * CHUNK) + 2 * CHUNK
                          + j * LANES, LANES)]
        plsc.addupdate_scatter(cnt, [rvec], ones)
        for e in range(LANES):
          r = rvec[e]
          i = j * LANES + e
          for v in range(nv2):
            sl = pl.ds(v * LANES, LANES)
            rv = plsc.bitcast(rtab[r, sl], jnp.bfloat16)
            sv = plsc.bitcast(buf[i, sl], jnp.bfloat16)
            dv = plsc.bitcast(buf[CHUNK + i, sl], jnp.bfloat16)
            m = jnp.maximum(sv + rv, zb)
            w = jnp.maximum(dv + rv, zb)
            ma, mb = plsc.unpack(m, format=plsc.PackFormat.INTERLEAVED)
            wa, wb = plsc.unpack(w, format=plsc.PackFormat.INTERLEAVED)
            a[2 * v] += ma
            a[2 * v + 1] += mb
            a[nvec + 2 * v] += wa
            a[nvec + 2 * v + 1] += wb
        return tuple(a)

      return lax.fori_loop(0, CHUNK // LANES, group_body, acc)

    gather(0, 0, bufP, semP)

    def seg_body(seg, carry):
      def pair_body(cp, acc):
        c0 = cp * 2
        c1 = c0 + 1
        gather(seg, c1, bufQ, semQ)
        wait1(bufP, semP)
        acc = compute(seg, c0, bufP, acc)

        @pl.when(c1 + 1 < nchunk)
        def _():
          gather(seg, c1 + 1, bufP, semP)

        wait1(bufQ, semQ)
        return compute(seg, c1, bufQ, acc)

      for t in range(RR // LANES):
        cnt[pl.ds(t * LANES, LANES)] = zvec
      acc0 = (zvec,) * (2 * nvec)
      acc = lax.fori_loop(0, nchunk // 2, pair_body, acc0)

      @pl.when(seg + 1 < S)
      def _():
        gather(seg + 1, 0, bufP, semP)
      ii2 = 2 * lax.iota(jnp.int32, LANES)
      for pool in range(2):
        for v in range(nv2):
          base = seg * 3 * H + 2 * pool * H + v * 2 * LANES
          plsc.store_scatter(res, [base + ii2],
                             acc[pool * nvec + 2 * v] * inv_e)
          plsc.store_scatter(res, [base + ii2 + 1],
                             acc[pool * nvec + 2 * v + 1] * inv_e)
      for v in range(nvec):
        res[pl.ds(seg * 3 * H + H + v * LANES, LANES)] = zvec
      cbase = pl.multiple_of((seg * B + wid) * RR, LANES)
      pltpu.sync_copy(cnt, cnt_hbm.at[pl.ds(cbase, RR)])
      return carry

    lax.fori_loop(0, S, seg_body, 0)
    for s in range(S):
      dst_off = pl.multiple_of(s * (B * 3 * H) + wid * (3 * H), 3 * H)
      pltpu.sync_copy(res.at[pl.ds(s * 3 * H, 3 * H)],
                      out_hbm.at[pl.ds(dst_off, 3 * H)])

  return k(eall, ent_tab, rel_tab_hbm)


def _tc_head(embed, cnt3, rtabf, W_ih, W_hh, bih, bhh, wr, br, prob, tl,
             S, B, H, E):
  inv_e = 1.0 / float(E)

  def body(embed_ref, cnt_ref, rtab_ref, wih_ref, whh_ref, bih_ref, bhh_ref,
           wr_ref, br_ref, prob_ref, tl_ref, out_ref):
    h = jnp.zeros((B, H), jnp.float32)
    wih = wih_ref[...]
    whh = whh_ref[...]
    bih_v = bih_ref[...]
    bhh_v = bhh_ref[...]
    rtw = jnp.dot(rtab_ref[...], wih[H:2 * H, :],
                  preferred_element_type=jnp.float32) * inv_e
    for s in range(S):
      x = embed_ref[s]
      gi = (jnp.dot(x, wih, preferred_element_type=jnp.float32)
            + jnp.dot(cnt_ref[s], rtw, preferred_element_type=jnp.float32)
            + bih_v)
      gh = jnp.dot(h, whh, preferred_element_type=jnp.float32) + bhh_v
      r = jax.nn.sigmoid(gi[:, 0:H] + gh[:, 0:H])
      z = jax.nn.sigmoid(gi[:, H:2 * H] + gh[:, H:2 * H])
      n = jnp.tanh(gi[:, 2 * H:3 * H] + r * gh[:, 2 * H:3 * H])
      h = (1.0 - z) * n + z * h
    logit = jnp.sum(h * wr_ref[...], axis=1, keepdims=True) + br_ref[0, 0]
    pred = jax.nn.sigmoid(logit)
    ii = lax.broadcasted_iota(jnp.int32, (B, prob_ref.shape[1]), 1)
    tmat = jnp.where(ii == tl_ref[...], prob_ref[...], 0.0)
    target = jnp.sum(tmat, axis=1, keepdims=True)
    eps = 1e-7
    p = jnp.clip(pred, eps, 1.0 - eps)
    li = target * jnp.log(p) + (1.0 - target) * jnp.log(1.0 - p)
    out_ref[...] = jnp.reshape(-jnp.mean(li), (1, 1))

  return pl.pallas_call(
      body,
      out_shape=jax.ShapeDtypeStruct((1, 1), jnp.float32),
  )(embed, cnt3, rtabf, W_ih, W_hh, bih, bhh, wr, br, prob, tl)


def kernel(t_list, true_prob_r, edge_src, edge_dst, edge_rel,
           ent_embeds, rel_embeds, W_ih, W_hh, b_ih, b_hh, W_r, b_r):
  B, S, E = edge_src.shape
  H = ent_embeds.shape[1]
  num_ents = ent_embeds.shape[0]
  num_rels = rel_embeds.shape[0]
  EP = ((E + CHUNK - 1) // CHUNK) * CHUNK

  def to_words(tab):
    b = jnp.concatenate(
        [tab, jnp.zeros((8, H), jnp.float32)], axis=0).astype(jnp.bfloat16)
    return lax.bitcast_convert_type(
        b.reshape(b.shape[0], H // 2, 2), jnp.int32)

  ent2 = to_words(ent_embeds)
  rel2 = to_words(rel_embeds)

  nchunk = EP // CHUNK

  def pad_edges(e, fill):
    e2 = e.reshape(B * S, E).astype(jnp.int32)
    pad = jnp.full((B * S, EP - E), fill, jnp.int32)
    return jnp.concatenate([e2, pad], axis=1).reshape(B * S, nchunk, CHUNK)

  esrc = pad_edges(edge_src, num_ents)
  edst = pad_edges(edge_dst, num_ents)
  erel = pad_edges(edge_rel, num_rels)
  eall = jnp.stack([esrc, edst, erel], axis=2).reshape(-1)

  embed_flat, cnt_flat = _sc_aggregate(eall, ent2, rel2, S, B, H, EP, E)
  embed = embed_flat.reshape(S, B, 3 * H)
  RR = 272
  cnt3 = cnt_flat.reshape(S, B, RR)
  rtabf = jnp.concatenate(
      [rel_embeds, jnp.zeros((RR - num_rels, H), jnp.float32)], axis=0)

  T = true_prob_r.shape[0]
  TP = ((T + H - 1) // H) * H
  prob = jnp.concatenate(
      [true_prob_r, jnp.zeros((TP - T,), jnp.float32)]).reshape(1, TP)
  tl = t_list.astype(jnp.int32).reshape(B, 1)

  loss = _tc_head(embed, cnt3, rtabf, W_ih, W_hh,
                  b_ih.reshape(1, 3 * H), b_hh.reshape(1, 3 * H),
                  W_r.reshape(1, H), b_r.reshape(1, 1),
                  prob, tl, S, B, H, E)
  return loss[0, 0]


# speedup vs baseline: 4.5345x; 2.2115x over previous
import functools

import jax
import jax.numpy as jnp
from jax import lax
from jax.experimental import pallas as pl
from jax.experimental.pallas import tpu as pltpu
from jax.experimental.pallas import tpu_sc as plsc

NC = 2
NS = 16
NW = NC * NS
LANES = 16
CHUNK = 128


def _sc_aggregate(eall, ent_tab, rel_tab_hbm, S, B, H, EP, E):
  nchunk = EP // CHUNK
  nvec = H // LANES
  rel_rows = rel_tab_hbm.shape[0]
  inv_e = 1.0 / float(E)
  idxseg = nchunk * 3 * CHUNK
  mesh = plsc.VectorSubcoreMesh(core_axis_name="c", subcore_axis_name="s")

  RR = 272

  @functools.partial(
      pl.kernel,
      out_type=(jax.ShapeDtypeStruct((S * B * 3 * H,), jnp.float32),
                jax.ShapeDtypeStruct((S * B * RR,), jnp.float32)),
      mesh=mesh,
      compiler_params=pltpu.CompilerParams(
          needs_layout_passes=False, use_tc_tiling_on_sc=False),
      scratch_types=[
          pltpu.VMEM_SHARED((10008, 64), jnp.int32),
          pltpu.VMEM((S * nchunk * 3 * CHUNK,), jnp.int16),
          pltpu.VMEM((3 * CHUNK,), jnp.int32),
          pltpu.VMEM((3 * CHUNK,), jnp.int32),
          pltpu.VMEM((rel_rows, H // 2), jnp.int32),
          pltpu.VMEM((272,), jnp.float32),
          pltpu.VMEM((2 * CHUNK, H // 2), jnp.int32),
          pltpu.VMEM((2 * CHUNK, H // 2), jnp.int32),
          pltpu.VMEM((S * 3 * H,), jnp.float32),
          pltpu.SemaphoreType.DMA,
          pltpu.SemaphoreType.DMA,
      ],
  )
  def k(eall_hbm, ent_hbm, rel_hbm, out_hbm, cnt_hbm,
        stab, idxb, idxP, idxQ, rtab, cnt, bufP, bufQ, res, semP, semQ):
    wid = lax.axis_index("s") * NC + lax.axis_index("c")
    zvec = jnp.zeros((LANES,), jnp.float32)
    nv2 = H // (2 * LANES)
    ones = jnp.ones((LANES,), jnp.float32)
    pltpu.sync_copy(rel_hbm, rtab)

    @pl.when(lax.axis_index("s") == 0)
    def _():
      pltpu.sync_copy(ent_hbm, stab)

    ibase = pl.multiple_of(wid * S * idxseg, CHUNK)
    pltpu.sync_copy(eall_hbm.at[pl.ds(ibase, S * idxseg)], idxb)
    plsc.subcore_barrier()

    def gather(seg, c, buf, idx32, sem):
      coff = seg * idxseg + c * (3 * CHUNK)
      for j in range(3 * CHUNK // (2 * LANES)):
        w = idxb[pl.ds(coff + j * 2 * LANES, 2 * LANES)]
        lo, hi = plsc.unpack(w, format=plsc.PackFormat.INTERLEAVED)
        idx32[pl.ds(j * 2 * LANES, LANES)] = lo
        idx32[pl.ds(j * 2 * LANES + LANES, LANES)] = hi
      pltpu.async_copy(stab.at[idx32.at[pl.ds(0, CHUNK)]],
                       buf.at[pl.ds(0, CHUNK)], sem)
      pltpu.async_copy(stab.at[idx32.at[pl.ds(CHUNK, CHUNK)]],
                       buf.at[pl.ds(CHUNK, CHUNK)], sem)

    def wait1(buf, sem):
      pltpu.make_async_copy(ent_hbm.at[pl.ds(0, 2 * CHUNK)], buf, sem).wait()

    def compute(seg, c, buf, idx32, acc):
      zb = jnp.zeros((2 * LANES,), jnp.bfloat16)

      def group_body(j, a):
        a = list(a)
        rvec = idx32[pl.ds(2 * CHUNK + j * LANES, LANES)]
        plsc.addupdate_scatter(cnt, [rvec], ones)
        for e in range(LANES):
          r = rvec[e]
          i = j * LANES + e
          for v in range(nv2):
            sl = pl.ds(v * LANES, LANES)
            rv = plsc.bitcast(rtab[r, sl], jnp.bfloat16)
            sv = plsc.bitcast(buf[i, sl], jnp.bfloat16)
            dv = plsc.bitcast(buf[CHUNK + i, sl], jnp.bfloat16)
            m = jnp.maximum(sv + rv, zb)
            w = jnp.maximum(dv + rv, zb)
            ma, mb = plsc.unpack(m, format=plsc.PackFormat.INTERLEAVED)
            wa, wb = plsc.unpack(w, format=plsc.PackFormat.INTERLEAVED)
            a[2 * v] += ma
            a[2 * v + 1] += mb
            a[nvec + 2 * v] += wa
            a[nvec + 2 * v + 1] += wb
        return tuple(a)

      return lax.fori_loop(0, CHUNK // LANES, group_body, acc)

    gather(0, 0, bufP, idxP, semP)

    def seg_body(seg, carry):
      def pair_body(cp, acc):
        c0 = cp * 2
        c1 = c0 + 1
        gather(seg, c1, bufQ, idxQ, semQ)
        wait1(bufP, semP)
        acc = compute(seg, c0, bufP, idxP, acc)

        @pl.when(c1 + 1 < nchunk)
        def _():
          gather(seg, c1 + 1, bufP, idxP, semP)

        wait1(bufQ, semQ)
        return compute(seg, c1, bufQ, idxQ, acc)

      for t in range(RR // LANES):
        cnt[pl.ds(t * LANES, LANES)] = zvec
      acc0 = (zvec,) * (2 * nvec)
      acc = lax.fori_loop(0, nchunk // 2, pair_body, acc0)

      @pl.when(seg + 1 < S)
      def _():
        gather(seg + 1, 0, bufP, idxP, semP)
      ii2 = 2 * lax.iota(jnp.int32, LANES)
      for pool in range(2):
        for v in range(nv2):
          base = seg * 3 * H + 2 * pool * H + v * 2 * LANES
          plsc.store_scatter(res, [base + ii2],
                             acc[pool * nvec + 2 * v] * inv_e)
          plsc.store_scatter(res, [base + ii2 + 1],
                             acc[pool * nvec + 2 * v + 1] * inv_e)
      for v in range(nvec):
        res[pl.ds(seg * 3 * H + H + v * LANES, LANES)] = zvec
      cbase = pl.multiple_of((seg * B + wid) * RR, LANES)
      pltpu.sync_copy(cnt, cnt_hbm.at[pl.ds(cbase, RR)])
      return carry

    lax.fori_loop(0, S, seg_body, 0)
    for s in range(S):
      dst_off = pl.multiple_of(s * (B * 3 * H) + wid * (3 * H), 3 * H)
      pltpu.sync_copy(res.at[pl.ds(s * 3 * H, 3 * H)],
                      out_hbm.at[pl.ds(dst_off, 3 * H)])

  return k(eall, ent_tab, rel_tab_hbm)


def _tc_head(embed, cnt3, rtabf, W_ih, W_hh, bih, bhh, wr, br, prob, tl,
             S, B, H, E):
  inv_e = 1.0 / float(E)

  def body(embed_ref, cnt_ref, rtab_ref, wih_ref, whh_ref, bih_ref, bhh_ref,
           wr_ref, br_ref, prob_ref, tl_ref, out_ref):
    h = jnp.zeros((B, H), jnp.float32)
    wih = wih_ref[...]
    whh = whh_ref[...]
    bih_v = bih_ref[...]
    bhh_v = bhh_ref[...]
    rtw = jnp.dot(rtab_ref[...], wih[H:2 * H, :],
                  preferred_element_type=jnp.float32) * inv_e
    for s in range(S):
      x = embed_ref[s]
      gi = (jnp.dot(x, wih, preferred_element_type=jnp.float32)
            + jnp.dot(cnt_ref[s], rtw, preferred_element_type=jnp.float32)
            + bih_v)
      gh = jnp.dot(h, whh, preferred_element_type=jnp.float32) + bhh_v
      r = jax.nn.sigmoid(gi[:, 0:H] + gh[:, 0:H])
      z = jax.nn.sigmoid(gi[:, H:2 * H] + gh[:, H:2 * H])
      n = jnp.tanh(gi[:, 2 * H:3 * H] + r * gh[:, 2 * H:3 * H])
      h = (1.0 - z) * n + z * h
    logit = jnp.sum(h * wr_ref[...], axis=1, keepdims=True) + br_ref[0, 0]
    pred = jax.nn.sigmoid(logit)
    ii = lax.broadcasted_iota(jnp.int32, (B, prob_ref.shape[1]), 1)
    tmat = jnp.where(ii == tl_ref[...], prob_ref[...], 0.0)
    target = jnp.sum(tmat, axis=1, keepdims=True)
    eps = 1e-7
    p = jnp.clip(pred, eps, 1.0 - eps)
    li = target * jnp.log(p) + (1.0 - target) * jnp.log(1.0 - p)
    out_ref[...] = jnp.reshape(-jnp.mean(li), (1, 1))

  return pl.pallas_call(
      body,
      out_shape=jax.ShapeDtypeStruct((1, 1), jnp.float32),
  )(embed, cnt3, rtabf, W_ih, W_hh, bih, bhh, wr, br, prob, tl)


def kernel(t_list, true_prob_r, edge_src, edge_dst, edge_rel,
           ent_embeds, rel_embeds, W_ih, W_hh, b_ih, b_hh, W_r, b_r):
  B, S, E = edge_src.shape
  H = ent_embeds.shape[1]
  num_ents = ent_embeds.shape[0]
  num_rels = rel_embeds.shape[0]
  EP = ((E + CHUNK - 1) // CHUNK) * CHUNK

  def to_words(tab):
    b = jnp.concatenate(
        [tab, jnp.zeros((8, H), jnp.float32)], axis=0).astype(jnp.bfloat16)
    return lax.bitcast_convert_type(
        b.reshape(b.shape[0], H // 2, 2), jnp.int32)

  ent2 = to_words(ent_embeds)
  rel2 = to_words(rel_embeds)

  nchunk = EP // CHUNK

  def pad_edges(e, fill):
    e2 = e.reshape(B * S, E).astype(jnp.int32)
    pad = jnp.full((B * S, EP - E), fill, jnp.int32)
    return jnp.concatenate([e2, pad], axis=1).reshape(B * S, nchunk, CHUNK)

  esrc = pad_edges(edge_src, num_ents)
  edst = pad_edges(edge_dst, num_ents)
  erel = pad_edges(edge_rel, num_rels)
  eall = jnp.stack([esrc, edst, erel], axis=2).reshape(-1).astype(jnp.int16)

  embed_flat, cnt_flat = _sc_aggregate(eall, ent2, rel2, S, B, H, EP, E)
  embed = embed_flat.reshape(S, B, 3 * H)
  RR = 272
  cnt3 = cnt_flat.reshape(S, B, RR)
  rtabf = jnp.concatenate(
      [rel_embeds, jnp.zeros((RR - num_rels, H), jnp.float32)], axis=0)

  T = true_prob_r.shape[0]
  TP = ((T + H - 1) // H) * H
  prob = jnp.concatenate(
      [true_prob_r, jnp.zeros((TP - T,), jnp.float32)]).reshape(1, TP)
  tl = t_list.astype(jnp.int32).reshape(B, 1)

  loss = _tc_head(embed, cnt3, rtabf, W_ih, W_hh,
                  b_ih.reshape(1, 3 * H), b_hh.reshape(1, 3 * H),
                  W_r.reshape(1, H), b_r.reshape(1, 1),
                  prob, tl, S, B, H, E)
  return loss[0, 0]


# speedup vs baseline: 4.6362x; 1.0224x over previous
import functools

import jax
import jax.numpy as jnp
from jax import lax
from jax.experimental import pallas as pl
from jax.experimental.pallas import tpu as pltpu
from jax.experimental.pallas import tpu_sc as plsc

NC = 2
NS = 16
NW = NC * NS
LANES = 16
CHUNK = 128


def _sc_aggregate(eall, ent_tab, rel_tab_hbm, S, B, H, EP, E):
  nchunk = EP // CHUNK
  nvec = H // LANES
  rel_rows = rel_tab_hbm.shape[0]
  inv_e = 1.0 / float(E)
  idxseg = nchunk * 3 * CHUNK
  mesh = plsc.VectorSubcoreMesh(core_axis_name="c", subcore_axis_name="s")

  RR = 272

  @functools.partial(
      pl.kernel,
      out_type=(jax.ShapeDtypeStruct((S * B * 3 * H,), jnp.float32),
                jax.ShapeDtypeStruct((S * B * RR,), jnp.float32)),
      mesh=mesh,
      compiler_params=pltpu.CompilerParams(
          needs_layout_passes=False, use_tc_tiling_on_sc=False),
      scratch_types=[
          pltpu.VMEM_SHARED((10008, 64), jnp.int32),
          pltpu.VMEM((S * nchunk * 3 * CHUNK,), jnp.int16),
          pltpu.VMEM((3 * CHUNK,), jnp.int32),
          pltpu.VMEM((3 * CHUNK,), jnp.int32),
          pltpu.VMEM((rel_rows, H // 2), jnp.int32),
          pltpu.VMEM((272,), jnp.float32),
          pltpu.VMEM((2 * CHUNK, H // 2), jnp.int32),
          pltpu.VMEM((2 * CHUNK, H // 2), jnp.int32),
          pltpu.VMEM((S * 3 * H,), jnp.float32),
          pltpu.SemaphoreType.DMA,
          pltpu.SemaphoreType.DMA,
      ],
  )
  def k(eall_hbm, ent_hbm, rel_hbm, out_hbm, cnt_hbm,
        stab, idxb, idxP, idxQ, rtab, cnt, bufP, bufQ, res, semP, semQ):
    wid = lax.axis_index("s") * NC + lax.axis_index("c")
    zvec = jnp.zeros((LANES,), jnp.float32)
    nv2 = H // (2 * LANES)
    ones = jnp.ones((LANES,), jnp.float32)
    pltpu.sync_copy(rel_hbm, rtab)

    @pl.when(lax.axis_index("s") == 0)
    def _():
      pltpu.sync_copy(ent_hbm, stab)

    ibase = pl.multiple_of(wid * S * idxseg, CHUNK)
    pltpu.sync_copy(eall_hbm.at[pl.ds(ibase, S * idxseg)], idxb)
    plsc.subcore_barrier()

    def gather(seg, c, buf, idx32, sem):
      coff = seg * idxseg + c * (3 * CHUNK)
      for j in range(3 * CHUNK // (2 * LANES)):
        w = idxb[pl.ds(coff + j * 2 * LANES, 2 * LANES)]
        lo, hi = plsc.unpack(w, format=plsc.PackFormat.INTERLEAVED)
        idx32[pl.ds(j * 2 * LANES, LANES)] = lo
        idx32[pl.ds(j * 2 * LANES + LANES, LANES)] = hi
      pltpu.async_copy(stab.at[idx32.at[pl.ds(0, CHUNK)]],
                       buf.at[pl.ds(0, CHUNK)], sem)
      pltpu.async_copy(stab.at[idx32.at[pl.ds(CHUNK, CHUNK)]],
                       buf.at[pl.ds(CHUNK, CHUNK)], sem)

    def wait1(buf, sem):
      pltpu.make_async_copy(ent_hbm.at[pl.ds(0, 2 * CHUNK)], buf, sem).wait()

    def compute(seg, c, buf, idx32, acc):
      zb = jnp.zeros((2 * LANES,), jnp.bfloat16)

      def group_body(j, a):
        a = list(a)
        rvec = idx32[pl.ds(2 * CHUNK + j * LANES, LANES)]
        plsc.addupdate_scatter(cnt, [rvec], ones)
        for e in range(LANES):
          r = rvec[e]
          i = j * LANES + e
          for v in range(nv2):
            sl = pl.ds(v * LANES, LANES)
            rv = plsc.bitcast(rtab[r, sl], jnp.bfloat16)
            sv = plsc.bitcast(buf[i, sl], jnp.bfloat16)
            dv = plsc.bitcast(buf[CHUNK + i, sl], jnp.bfloat16)
            m = jnp.maximum(sv + rv, zb)
            w = jnp.maximum(dv + rv, zb)
            ma, mb = plsc.unpack(m, format=plsc.PackFormat.INTERLEAVED)
            wa, wb = plsc.unpack(w, format=plsc.PackFormat.INTERLEAVED)
            a[2 * v] += ma
            a[2 * v + 1] += mb
            a[nvec + 2 * v] += wa
            a[nvec + 2 * v + 1] += wb
        return tuple(a)

      return lax.fori_loop(0, CHUNK // LANES, group_body, acc, unroll=2)

    gather(0, 0, bufP, idxP, semP)

    def seg_body(seg, carry):
      def pair_body(cp, acc):
        c0 = cp * 2
        c1 = c0 + 1
        gather(seg, c1, bufQ, idxQ, semQ)
        wait1(bufP, semP)
        acc = compute(seg, c0, bufP, idxP, acc)

        @pl.when(c1 + 1 < nchunk)
        def _():
          gather(seg, c1 + 1, bufP, idxP, semP)

        wait1(bufQ, semQ)
        return compute(seg, c1, bufQ, idxQ, acc)

      for t in range(RR // LANES):
        cnt[pl.ds(t * LANES, LANES)] = zvec
      acc0 = (zvec,) * (2 * nvec)
      acc = lax.fori_loop(0, nchunk // 2, pair_body, acc0)

      @pl.when(seg + 1 < S)
      def _():
        gather(seg + 1, 0, bufP, idxP, semP)
      ii2 = 2 * lax.iota(jnp.int32, LANES)
      for pool in range(2):
        for v in range(nv2):
          base = seg * 3 * H + 2 * pool * H + v * 2 * LANES
          plsc.store_scatter(res, [base + ii2],
                             acc[pool * nvec + 2 * v] * inv_e)
          plsc.store_scatter(res, [base + ii2 + 1],
                             acc[pool * nvec + 2 * v + 1] * inv_e)
      for v in range(nvec):
        res[pl.ds(seg * 3 * H + H + v * LANES, LANES)] = zvec
      cbase = pl.multiple_of((seg * B + wid) * RR, LANES)
      pltpu.sync_copy(cnt, cnt_hbm.at[pl.ds(cbase, RR)])
      return carry

    lax.fori_loop(0, S, seg_body, 0)
    for s in range(S):
      dst_off = pl.multiple_of(s * (B * 3 * H) + wid * (3 * H), 3 * H)
      pltpu.sync_copy(res.at[pl.ds(s * 3 * H, 3 * H)],
                      out_hbm.at[pl.ds(dst_off, 3 * H)])

  return k(eall, ent_tab, rel_tab_hbm)


def _tc_head(embed, cnt3, rtabf, W_ih, W_hh, bih, bhh, wr, br, prob, tl,
             S, B, H, E):
  inv_e = 1.0 / float(E)

  def body(embed_ref, cnt_ref, rtab_ref, wih_ref, whh_ref, bih_ref, bhh_ref,
           wr_ref, br_ref, prob_ref, tl_ref, out_ref):
    h = jnp.zeros((B, H), jnp.float32)
    wih = wih_ref[...]
    whh = whh_ref[...]
    bih_v = bih_ref[...]
    bhh_v = bhh_ref[...]
    rtw = jnp.dot(rtab_ref[...], wih[H:2 * H, :],
                  preferred_element_type=jnp.float32) * inv_e
    for s in range(S):
      x = embed_ref[s]
      gi = (jnp.dot(x, wih, preferred_element_type=jnp.float32)
            + jnp.dot(cnt_ref[s], rtw, preferred_element_type=jnp.float32)
            + bih_v)
      gh = jnp.dot(h, whh, preferred_element_type=jnp.float32) + bhh_v
      r = jax.nn.sigmoid(gi[:, 0:H] + gh[:, 0:H])
      z = jax.nn.sigmoid(gi[:, H:2 * H] + gh[:, H:2 * H])
      n = jnp.tanh(gi[:, 2 * H:3 * H] + r * gh[:, 2 * H:3 * H])
      h = (1.0 - z) * n + z * h
    logit = jnp.sum(h * wr_ref[...], axis=1, keepdims=True) + br_ref[0, 0]
    pred = jax.nn.sigmoid(logit)
    ii = lax.broadcasted_iota(jnp.int32, (B, prob_ref.shape[1]), 1)
    tmat = jnp.where(ii == tl_ref[...], prob_ref[...], 0.0)
    target = jnp.sum(tmat, axis=1, keepdims=True)
    eps = 1e-7
    p = jnp.clip(pred, eps, 1.0 - eps)
    li = target * jnp.log(p) + (1.0 - target) * jnp.log(1.0 - p)
    out_ref[...] = jnp.reshape(-jnp.mean(li), (1, 1))

  return pl.pallas_call(
      body,
      out_shape=jax.ShapeDtypeStruct((1, 1), jnp.float32),
  )(embed, cnt3, rtabf, W_ih, W_hh, bih, bhh, wr, br, prob, tl)


def kernel(t_list, true_prob_r, edge_src, edge_dst, edge_rel,
           ent_embeds, rel_embeds, W_ih, W_hh, b_ih, b_hh, W_r, b_r):
  B, S, E = edge_src.shape
  H = ent_embeds.shape[1]
  num_ents = ent_embeds.shape[0]
  num_rels = rel_embeds.shape[0]
  EP = ((E + CHUNK - 1) // CHUNK) * CHUNK

  def to_words(tab):
    b = jnp.concatenate(
        [tab, jnp.zeros((8, H), jnp.float32)], axis=0).astype(jnp.bfloat16)
    return lax.bitcast_convert_type(
        b.reshape(b.shape[0], H // 2, 2), jnp.int32)

  ent2 = to_words(ent_embeds)
  rel2 = to_words(rel_embeds)

  nchunk = EP // CHUNK

  def pad_edges(e, fill):
    e2 = e.reshape(B * S, E).astype(jnp.int32)
    pad = jnp.full((B * S, EP - E), fill, jnp.int32)
    return jnp.concatenate([e2, pad], axis=1).reshape(B * S, nchunk, CHUNK)

  esrc = pad_edges(edge_src, num_ents)
  edst = pad_edges(edge_dst, num_ents)
  erel = pad_edges(edge_rel, num_rels)
  eall = jnp.stack([esrc, edst, erel], axis=2).reshape(-1).astype(jnp.int16)

  embed_flat, cnt_flat = _sc_aggregate(eall, ent2, rel2, S, B, H, EP, E)
  embed = embed_flat.reshape(S, B, 3 * H)
  RR = 272
  cnt3 = cnt_flat.reshape(S, B, RR)
  rtabf = jnp.concatenate(
      [rel_embeds, jnp.zeros((RR - num_rels, H), jnp.float32)], axis=0)

  T = true_prob_r.shape[0]
  TP = ((T + H - 1) // H) * H
  prob = jnp.concatenate(
      [true_prob_r, jnp.zeros((TP - T,), jnp.float32)]).reshape(1, TP)
  tl = t_list.astype(jnp.int32).reshape(B, 1)

  loss = _tc_head(embed, cnt3, rtabf, W_ih, W_hh,
                  b_ih.reshape(1, 3 * H), b_hh.reshape(1, 3 * H),
                  W_r.reshape(1, H), b_r.reshape(1, 1),
                  prob, tl, S, B, H, E)
  return loss[0, 0]
